# Initial kernel scaffold; baseline (speedup 1.0000x reference)
import jax
import jax.numpy as jnp
from jax.experimental import pallas as pl


def kernel(indices, pe):
    raise NotImplementedError("write your pallas kernel here")


# speedup vs baseline: 4.6331x; 4.6331x over previous
import functools

import jax
import jax.numpy as jnp
from jax import lax
from jax.experimental import pallas as pl
from jax.experimental.pallas import tpu as pltpu
from jax.experimental.pallas import tpu_sc as plsc

_info = plsc.get_sparse_core_info()
_NC, _NS = _info.num_cores, _info.num_subcores
_NW = _NC * _NS


@functools.lru_cache(maxsize=None)
def _make_gather(n_rows, d_model, chunk):
    assert n_rows % (_NW * chunk) == 0
    bpw = n_rows // _NW
    n_chunks = bpw // chunk

    mesh = plsc.VectorSubcoreMesh(core_axis_name="c", subcore_axis_name="s")

    @functools.partial(
        pl.kernel,
        out_type=jax.ShapeDtypeStruct((n_rows, d_model), jnp.float32),
        mesh=mesh,
        scratch_types=[
            pltpu.VMEM((chunk,), jnp.int32),
            pltpu.VMEM((chunk, d_model), jnp.float32),
            pltpu.SemaphoreType.DMA,
        ],
        compiler_params=pltpu.CompilerParams(use_tc_tiling_on_sc=False),
    )
    def gather(idx_hbm, table_hbm, out_hbm, idx_v, rows_v, sem):
        wid = lax.axis_index("s") * _NC + lax.axis_index("c")
        base = wid * bpw

        def body(g, carry):
            off = base + g * chunk
            pltpu.sync_copy(idx_hbm.at[pl.ds(off, chunk)], idx_v)
            pltpu.async_copy(table_hbm.at[idx_v], rows_v, sem).wait()
            pltpu.sync_copy(rows_v, out_hbm.at[pl.ds(off, chunk)])
            return carry

        lax.fori_loop(0, n_chunks, body, 0)

    return gather


def kernel(indices, pe):
    b, l = indices.shape
    d_model = pe.shape[1]
    flat = indices.reshape(-1)
    out = _make_gather(b * l, d_model, 512)(flat, pe)
    return out.reshape(b, l, d_model)


# speedup vs baseline: 4.9796x; 1.0748x over previous
import functools

import jax
import jax.numpy as jnp
from jax import lax
from jax.experimental import pallas as pl
from jax.experimental.pallas import tpu as pltpu
from jax.experimental.pallas import tpu_sc as plsc

_info = plsc.get_sparse_core_info()
_NC, _NS = _info.num_cores, _info.num_subcores
_NW = _NC * _NS


@functools.lru_cache(maxsize=None)
def _make_gather(n_rows, d_model, chunk):
    assert n_rows % (_NW * chunk) == 0
    bpw = n_rows // _NW
    n_chunks = bpw // chunk
    assert n_chunks >= 2 and n_chunks % 2 == 0

    mesh = plsc.VectorSubcoreMesh(core_axis_name="c", subcore_axis_name="s")

    @functools.partial(
        pl.kernel,
        out_type=jax.ShapeDtypeStruct((n_rows, d_model), jnp.float32),
        mesh=mesh,
        scratch_types=[
            pltpu.VMEM((2, chunk), jnp.int32),
            pltpu.VMEM((2, chunk, d_model), jnp.float32),
            pltpu.SemaphoreType.DMA,
            pltpu.SemaphoreType.DMA,
            pltpu.SemaphoreType.DMA,
            pltpu.SemaphoreType.DMA,
        ],
        compiler_params=pltpu.CompilerParams(use_tc_tiling_on_sc=False),
    )
    def gather(idx_hbm, table_hbm, out_hbm, idx_v, rows_v, g0, g1, s0, s1):
        gsems = (g0, g1)
        ssems = (s0, s1)
        wid = lax.axis_index("s") * _NC + lax.axis_index("c")
        base = wid * bpw

        def load_idx(g, b):
            pltpu.sync_copy(idx_hbm.at[pl.ds(base + g * chunk, chunk)],
                            idx_v.at[b])

        def start_gather(b):
            pltpu.async_copy(table_hbm.at[idx_v.at[b]], rows_v.at[b], gsems[b])

        def wait_gather(b):
            pltpu.make_async_copy(table_hbm.at[idx_v.at[b]], rows_v.at[b],
                                  gsems[b]).wait()

        def start_store(g, b):
            pltpu.async_copy(rows_v.at[b],
                             out_hbm.at[pl.ds(base + g * chunk, chunk)],
                             ssems[b])

        def wait_store(b):
            pltpu.make_async_copy(rows_v.at[b],
                                  out_hbm.at[pl.ds(base, chunk)],
                                  ssems[b]).wait()

        load_idx(0, 0)
        start_gather(0)

        def step(gg, carry):
            for b in (0, 1):
                g = gg * 2 + b
                nb = 1 - b

                @pl.when(g + 1 < n_chunks)
                def _prefetch():
                    load_idx(g + 1, nb)

                    @pl.when(g >= 1)
                    def _reclaim():
                        wait_store(nb)

                    start_gather(nb)

                wait_gather(b)
                start_store(g, b)
            return carry

        lax.fori_loop(0, n_chunks // 2, step, 0)
        wait_store(0)
        wait_store(1)

    return gather


def kernel(indices, pe):
    b, l = indices.shape
    d_model = pe.shape[1]
    flat = indices.reshape(-1)
    out = _make_gather(b * l, d_model, 512)(flat, pe)
    return out.reshape(b, l, d_model)
